# baseline (speedup 1.0000x reference)
import jax
import jax.numpy as jnp
from jax import lax
from jax.experimental import pallas as pl
from jax.experimental.pallas import tpu as pltpu
from jax.experimental.pallas import tpu_sc as plsc

B = 16384
D = 64
DP = 128
NE = 1_000_000
NR = 1000
NC = 2
NS = 16
NW = NC * NS
BPW = B // NW
G = 128
NG_FULL = NE // G
TAILN = NE - NG_FULL * G
GPW = 248
KG = 4
NTASK = 4
NIDX = NTASK * B
SELCAP = 4096
CAPG = 32
ROWCAP = 128
FLUSH_HI = 81
OUTROWS = 6 * B + DP
DUMP = 6 * B


def _sel_scan2(buf_a, buf_b, ta, tb, cbase, glo, ghi, sel_idx, sel_dst, offs):
    lanes = lax.iota(jnp.int32, 16)

    HALF = SELCAP // 2

    def chunk(c, carry):
        oa, ob = carry
        pa, pb = [], []
        for buf, t, off, lim in ((buf_a, ta, oa, HALF - 16),
                                 (buf_b, tb, ob, SELCAP - 16)):
            o = jnp.minimum(off, lim)
            v = buf[pl.ds(c * 16, 16)]
            g = lax.shift_right_logical(v, 7)
            m = (g >= glo) & (g < ghi)
            plsc.store_compressed(sel_idx.at[pl.ds(o, 16)], v, mask=m)
            plsc.store_compressed(sel_dst.at[pl.ds(o, 16)],
                                  t * B + cbase + c * 16 + lanes, mask=m)
            (pa if t == ta else pb).append(
                plsc.all_reduce_population_count(m)[0])
        oa = jnp.minimum(oa + pa[0], HALF - 16)
        ob = jnp.minimum(ob + pb[0], SELCAP - 16)
        return (oa, ob)

    return pl.loop(0, B // 32, init_carry=offs)(chunk)


def _gather_body(h_i, r_i, t_i, he_i, re_i, te_i,
                 ent_t, rel128, tail128,
                 out,
                 idx_b, idx_b2, sel_idx, sel_dst, bk_idx, bk_dst,
                 gbuf0, gbuf1, rowbuf, destv, cnt_s,
                 sem, gsem0, gsem1, ssem):
    wid = lax.axis_index("s") * NC + lax.axis_index("c")
    base = wid * BPW
    glo = wid * GPW
    ghi = jnp.minimum(glo + GPW, NG_FULL)
    lanes = lax.iota(jnp.int32, 16)

    for slot, idx_hbm, table in ((4, r_i, rel128), (5, re_i, rel128)):
        pltpu.sync_copy(idx_hbm.at[pl.ds(base, BPW)], idx_b.at[pl.ds(0, BPW)])
        for q in range(4):
            qb = q * (BPW // 4)
            pltpu.async_copy(
                table.at[idx_b.at[pl.ds(qb, BPW // 4)]],
                rowbuf.at[pl.ds(0, BPW // 4)], sem).wait()
            pltpu.sync_copy(rowbuf.at[pl.ds(0, BPW // 4)],
                            out.at[pl.ds(slot * B + base + qb, BPW // 4)])

    @pl.loop(0, SELCAP // 16)
    def _prefill(c):
        sel_idx[pl.ds(c * 16, 16)] = jnp.full((16,), glo * G, jnp.int32)
        sel_dst[pl.ds(c * 16, 16)] = jnp.full((16,), DUMP, jnp.int32)

    HALF = SELCAP // 2
    offs = (0, HALF)
    for ta, tb, a_i, b_i in ((0, 1, h_i, t_i), (2, 3, he_i, te_i)):
        for hb in range(2):
            pltpu.sync_copy(a_i.at[pl.ds(hb * (B // 2), B // 2)], idx_b)
            pltpu.sync_copy(b_i.at[pl.ds(hb * (B // 2), B // 2)], idx_b2)
            offs = _sel_scan2(idx_b, idx_b2, ta, tb, hb * (B // 2), glo, ghi,
                              sel_idx, sel_dst, offs)
    off_a, off_b = offs

    @pl.when(wid == NW - 1)
    def _tail():
        toff = 0
        for t, idx_hbm in enumerate((h_i, t_i, he_i, te_i)):
            for hb in range(2):
                pltpu.sync_copy(idx_hbm.at[pl.ds(hb * (B // 2), B // 2)], idx_b)

                def tchunk(c, o, t=t, hb=hb):
                    oc = jnp.minimum(o, ROWCAP - 16)
                    v = idx_b[pl.ds(c * 16, 16)]
                    m = v >= NG_FULL * G
                    plsc.store_compressed(bk_idx.at[pl.ds(oc, 16)],
                                          v - NG_FULL * G, mask=m)
                    plsc.store_compressed(
                        bk_dst.at[pl.ds(oc, 16)],
                        t * B + hb * (B // 2) + c * 16 + lanes, mask=m)
                    pop = plsc.all_reduce_population_count(m)[0]
                    return jnp.minimum(o + pop, ROWCAP - 16)

                toff = pl.loop(0, B // 32, init_carry=toff)(tchunk)
        nt = toff

        @pl.loop(0, ROWCAP // 16)
        def _pad(c):
            v = bk_idx[pl.ds(c * 16, 16)]
            d_ = bk_dst[pl.ds(c * 16, 16)]
            m = (c * 16 + lanes) < nt
            bk_idx[pl.ds(c * 16, 16)] = jnp.where(m, v, 0)
            destv[pl.ds(c * 16, 16)] = jnp.where(m, d_, DUMP)

        pltpu.async_copy(tail128.at[bk_idx.at[pl.ds(0, ROWCAP)]],
                         rowbuf, sem).wait()
        pltpu.async_copy(rowbuf, out.at[destv], sem).wait()

    @pl.loop(0, GPW)
    def _zero(g):
        cnt_s[g] = 0

    @pl.loop(0, (jnp.maximum(off_a, 16) + 15) // 16)
    def _bucket(c):
        v = sel_idx[pl.ds(c * 16, 16)]
        d_ = sel_dst[pl.ds(c * 16, 16)]
        for lane in range(16):
            r = v[lane]
            dd = d_[lane]
            gl = lax.shift_right_logical(r, 7) - glo
            ccur = cnt_s[gl]
            slot = gl * CAPG + ccur
            plsc.store_scatter(
                bk_idx, [jnp.full((16,), slot, jnp.int32)],
                jnp.full((16,), r & (G - 1), jnp.int32), mask=lanes == 0)
            plsc.store_scatter(
                bk_dst, [jnp.full((16,), slot, jnp.int32)],
                jnp.full((16,), dd, jnp.int32), mask=lanes == 0)
            cnt_s[gl] = jnp.minimum(ccur + 1, CAPG - 1)

    @pl.loop(HALF // 16, (jnp.maximum(off_b, HALF + 16) + 15) // 16)
    def _bucket2(c):
        v = sel_idx[pl.ds(c * 16, 16)]
        d_ = sel_dst[pl.ds(c * 16, 16)]
        for lane in range(16):
            r = v[lane]
            dd = d_[lane]
            gl = lax.shift_right_logical(r, 7) - glo
            ccur = cnt_s[gl]
            slot = gl * CAPG + ccur
            plsc.store_scatter(
                bk_idx, [jnp.full((16,), slot, jnp.int32)],
                jnp.full((16,), r & (G - 1), jnp.int32), mask=lanes == 0)
            plsc.store_scatter(
                bk_dst, [jnp.full((16,), slot, jnp.int32)],
                jnp.full((16,), dd, jnp.int32), mask=lanes == 0)
            cnt_s[gl] = jnp.minimum(ccur + 1, CAPG - 1)

    @pl.loop(0, ROWCAP // 16)
    def _dfill(c):
        destv[pl.ds(c * 16, 16)] = jnp.full((16,), DUMP, jnp.int32)

    nfull = jnp.minimum(glo + GPW, NG_FULL) - glo
    nsup = nfull // KG

    def fetch_super(sidx, gb, gsm):
        c0 = (glo + sidx * KG) * G
        for b in range(8):
            pltpu.async_copy(
                ent_t.at[pl.ds(b * 8, 8), pl.ds(c0, KG * G)],
                gb.at[pl.ds(b * 8, 8), :], gsm)

    def wait_super(gb, gsm):
        pltpu.make_async_copy(
            ent_t.at[:, pl.ds(0, KG * G)], gb, gsm).wait()

    fetch_super(0, gbuf0, gsem0)
    fetch_super(1, gbuf1, gsem1)

    def do_super(p, nrow):
        def body(gb, gsm, cur):
            wait_super(gb, gsm)
            nr = nrow
            for j in range(KG):
                k = p * KG + j
                cnt = cnt_s[k]
                coff = j * G

                def hit_chunk(cb, nr2, k=k, cnt=cnt, coff=coff):
                    bbase = k * CAPG + cb * 16
                    rloc = bk_idx[pl.ds(bbase, 16)] + coff
                    dst16 = bk_dst[pl.ds(bbase, 16)]
                    m = (cb * 16 + lanes) < cnt
                    rloc = jnp.where(m, rloc, 0)
                    destv[pl.ds(nr2, 16)] = jnp.where(m, dst16, DUMP)
                    slots = nr2 + lanes
                    for d0 in range(0, D, 4):
                        xs = [plsc.load_gather(
                            gb, [jnp.full((16,), d0 + i, jnp.int32), rloc],
                            mask=m) for i in range(4)]
                        for i in range(4):
                            plsc.store_scatter(
                                rowbuf,
                                [slots, jnp.full((16,), d0 + i, jnp.int32)],
                                xs[i], mask=m)
                    return nr2 + jnp.minimum(cnt - cb * 16, 16)

                nr = pl.loop(0, (cnt + 15) // 16, init_carry=nr)(hit_chunk)

                def flush(nr=nr):
                    @pl.loop(0, ROWCAP // 16)
                    def _san(c):
                        dv = destv[pl.ds(c * 16, 16)]
                        m = (c * 16 + lanes) < nr
                        destv[pl.ds(c * 16, 16)] = jnp.where(m, dv, DUMP)

                    pltpu.async_copy(rowbuf, out.at[destv], ssem).wait()
                    return 0

                nr = lax.cond(nr >= FLUSH_HI, flush, lambda nr=nr: nr)

            @pl.when(p + 2 < nsup)
            def _pf():
                fetch_super(p + 2, gb, gsm)

            return nr

        return lax.cond(p % 2 == 0,
                        lambda: body(gbuf0, gsem0, 0),
                        lambda: body(gbuf1, gsem1, 1))

    nrow_end = pl.loop(0, nsup, init_carry=0)(do_super)

    @pl.loop(0, ROWCAP // 16)
    def _san2(c):
        dv = destv[pl.ds(c * 16, 16)]
        m = (c * 16 + lanes) < nrow_end
        destv[pl.ds(c * 16, 16)] = jnp.where(m, dv, DUMP)

    pltpu.async_copy(rowbuf, out.at[destv], ssem).wait()


_mesh = plsc.VectorSubcoreMesh(core_axis_name="c", subcore_axis_name="s")

_gather = pl.kernel(
    _gather_body,
    mesh=_mesh,
    out_type=jax.ShapeDtypeStruct((OUTROWS, DP), jnp.float32),
    scratch_types=[
        pltpu.VMEM((B // 2,), jnp.int32),
        pltpu.VMEM((B // 2,), jnp.int32),
        pltpu.VMEM((SELCAP,), jnp.int32),
        pltpu.VMEM((SELCAP,), jnp.int32),
        pltpu.VMEM((GPW * CAPG,), jnp.int32),
        pltpu.VMEM((GPW * CAPG,), jnp.int32),
        pltpu.VMEM((D, KG * G), jnp.float32),
        pltpu.VMEM((D, KG * G), jnp.float32),
        pltpu.VMEM((ROWCAP, DP), jnp.float32),
        pltpu.VMEM((ROWCAP,), jnp.int32),
        pltpu.SMEM((GPW,), jnp.int32),
        pltpu.SemaphoreType.DMA,
        pltpu.SemaphoreType.DMA,
        pltpu.SemaphoreType.DMA,
        pltpu.SemaphoreType.DMA,
    ],
    compiler_params=pltpu.CompilerParams(use_tc_tiling_on_sc=True,
                                         needs_layout_passes=False),
)


def kernel(pos_head, pos_rel, pos_tail, pos_head_exp, pos_rel_exp,
           pos_tail_exp, entity_table, rel_table):
    idxs = [jnp.asarray(x, jnp.int32) for x in
            (pos_head, pos_rel, pos_tail, pos_head_exp, pos_rel_exp, pos_tail_exp)]
    rel128 = jnp.pad(rel_table, ((0, 0), (0, DP - D)))
    tail128 = jnp.pad(entity_table[NG_FULL * G:], ((0, 0), (0, DP - D)))
    out = _gather(*idxs, entity_table.T, rel128, tail128)
    s = [out[k * B:(k + 1) * B, :D] for k in range(6)]
    return (s[0], s[4], s[1], s[2], s[5], s[3])


# speedup vs baseline: 2.7132x; 2.7132x over previous
import jax
import jax.numpy as jnp
from jax import lax
from jax.experimental import pallas as pl
from jax.experimental.pallas import tpu as pltpu
from jax.experimental.pallas import tpu_sc as plsc

B = 16384
D = 64
NC = 2
NS = 16
NW = NC * NS
BPW = B // NW


def _gather6_body(h_i, r_i, t_i, he_i, re_i, te_i, ent, rel,
                  o0, o1, o2, o3, o4, o5,
                  idx_v, rows_v, sem):
    wid = lax.axis_index("s") * NC + lax.axis_index("c")
    base = wid * BPW
    tasks = ((h_i, ent, o0), (r_i, rel, o1), (t_i, ent, o2),
             (he_i, ent, o3), (re_i, rel, o4), (te_i, ent, o5))
    for idx_hbm, table, out_hbm in tasks:
        pltpu.sync_copy(idx_hbm.at[pl.ds(base, BPW)], idx_v)
        pltpu.async_copy(table.at[idx_v], rows_v, sem).wait()
        pltpu.sync_copy(rows_v, out_hbm.at[pl.ds(base, BPW)])


_mesh = plsc.VectorSubcoreMesh(core_axis_name="c", subcore_axis_name="s")

_gather6 = pl.kernel(
    _gather6_body,
    mesh=_mesh,
    out_type=tuple(jax.ShapeDtypeStruct((B, D), jnp.float32) for _ in range(6)),
    scratch_types=[
        pltpu.VMEM((BPW,), jnp.int32),
        pltpu.VMEM((BPW, D), jnp.float32),
        pltpu.SemaphoreType.DMA,
    ],
    compiler_params=pltpu.CompilerParams(use_tc_tiling_on_sc=False),
)


def kernel(pos_head, pos_rel, pos_tail, pos_head_exp, pos_rel_exp,
           pos_tail_exp, entity_table, rel_table):
    idxs = [jnp.asarray(x, jnp.int32) for x in
            (pos_head, pos_rel, pos_tail, pos_head_exp, pos_rel_exp, pos_tail_exp)]
    return _gather6(*idxs, entity_table, rel_table)
